# baseline (speedup 1.0000x reference)
import functools

import jax
import jax.numpy as jnp
from jax import lax
from jax.experimental import pallas as pl
from jax.experimental.pallas import tpu as pltpu
from jax.experimental.pallas import tpu_sc as plsc

_NC = 2
_NS = 16
_NW = _NC * _NS


@functools.lru_cache(maxsize=None)
def _make_drow_gather(hist: int, batch: int, vocab: int, d: int):
    d_per_w = d // _NW
    n_pairs = hist // 2
    nv = batch // 16
    assert d % _NW == 0 and hist % 2 == 0 and batch % 16 == 0

    mesh = plsc.VectorSubcoreMesh(core_axis_name="c", subcore_axis_name="s")

    @functools.partial(
        pl.kernel,
        mesh=mesh,
        compiler_params=pltpu.CompilerParams(
            use_tc_tiling_on_sc=False, needs_layout_passes=False
        ),
        out_type=jax.ShapeDtypeStruct((hist, d, batch), jnp.float32),
        scratch_types=[
            pltpu.VMEM((vocab,), jnp.float32),
            pltpu.VMEM((batch,), jnp.int32),
            pltpu.VMEM((batch,), jnp.float32),
            pltpu.VMEM((batch,), jnp.float32),
            pltpu.SemaphoreType.DMA,
            pltpu.SemaphoreType.DMA,
        ],
    )
    def drow_kernel(idx_hbm, tab_hbm, out_hbm, row, ibuf, ob0, ob1,
                    osem0, osem1):
        wid = lax.axis_index("s") * _NC + lax.axis_index("c")

        for dpass in range(d_per_w):
            dd = wid * d_per_w + dpass
            pltpu.sync_copy(tab_hbm.at[dd], row)

            def fire_out(h, ob, osem):
                pltpu.async_copy(ob, out_hbm.at[h, dd], osem)

            def wait_out(h, ob, osem):
                pltpu.make_async_copy(ob, out_hbm.at[h, dd], osem).wait()

            def gather_row(h, ob):
                pltpu.sync_copy(idx_hbm.at[h], ibuf)
                for v in range(nv):
                    iv = ibuf[pl.ds(16 * v, 16)]
                    ob[pl.ds(16 * v, 16)] = plsc.load_gather(row, [iv])

            def body(t, carry):
                h0 = 2 * t
                h1 = h0 + 1

                @pl.when(t >= 1)
                def _wait_prev_out0():
                    wait_out(h0 - 2, ob0, osem0)

                gather_row(h0, ob0)
                fire_out(h0, ob0, osem0)

                @pl.when(t >= 1)
                def _wait_prev_out1():
                    wait_out(h1 - 2, ob1, osem1)

                gather_row(h1, ob1)
                fire_out(h1, ob1, osem1)
                return carry

            lax.fori_loop(0, n_pairs, body, 0)
            wait_out(hist - 2, ob0, osem0)
            wait_out(hist - 1, ob1, osem1)

    return drow_kernel


def kernel(indices, table):
    batch, hist = indices.shape
    vocab, d = table.shape
    idx_t = jnp.transpose(indices).astype(jnp.int32)
    tab_t = jnp.transpose(table)
    out_t = _make_drow_gather(hist, batch, vocab, d)(idx_t, tab_t)
    return jnp.transpose(out_t, (2, 0, 1))


# speedup vs baseline: 1.4934x; 1.4934x over previous
import functools

import jax
import jax.numpy as jnp
from jax import lax
from jax.experimental import pallas as pl
from jax.experimental.pallas import tpu as pltpu
from jax.experimental.pallas import tpu_sc as plsc

_NC = 2
_NS = 16
_NW = _NC * _NS


@functools.lru_cache(maxsize=None)
def _make_drow_gather(hist: int, batch: int, vocab: int, d: int):
    d_per_w = d // _NW
    n_pairs = hist // 2
    nv = batch // 16
    assert d % _NW == 0 and hist % 2 == 0 and batch % 16 == 0

    mesh = plsc.VectorSubcoreMesh(core_axis_name="c", subcore_axis_name="s")

    @functools.partial(
        pl.kernel,
        mesh=mesh,
        compiler_params=pltpu.CompilerParams(
            use_tc_tiling_on_sc=False, needs_layout_passes=False
        ),
        out_type=jax.ShapeDtypeStruct((hist, d, batch), jnp.float32),
        scratch_types=[
            pltpu.VMEM((vocab,), jnp.float32),
            pltpu.VMEM((batch,), jnp.int32),
            pltpu.VMEM((batch,), jnp.float32),
            pltpu.VMEM((batch,), jnp.float32),
            pltpu.SemaphoreType.DMA,
            pltpu.SemaphoreType.DMA,
        ],
    )
    def drow_kernel(idx_hbm, tab_hbm, out_hbm, row, ibuf, ob0, ob1,
                    osem0, osem1):
        wid = lax.axis_index("s") * _NC + lax.axis_index("c")

        for dpass in range(d_per_w):
            dd = wid * d_per_w + dpass
            pltpu.sync_copy(tab_hbm.at[dd], row)

            def fire_out(h, ob, osem):
                pltpu.async_copy(ob, out_hbm.at[h, dd], osem)

            def wait_out(h, ob, osem):
                pltpu.make_async_copy(ob, out_hbm.at[h, dd], osem).wait()

            def gather_row(h, ob):
                pltpu.sync_copy(idx_hbm.at[h], ibuf)

                @plsc.parallel_loop(0, nv, unroll=8)
                def _gather(v):
                    iv = ibuf[pl.ds(v * 16, 16)]
                    ob[pl.ds(v * 16, 16)] = plsc.load_gather(row, [iv])

            def body(t, carry):
                h0 = 2 * t
                h1 = h0 + 1

                @pl.when(t >= 1)
                def _wait_prev_out0():
                    wait_out(h0 - 2, ob0, osem0)

                gather_row(h0, ob0)
                fire_out(h0, ob0, osem0)

                @pl.when(t >= 1)
                def _wait_prev_out1():
                    wait_out(h1 - 2, ob1, osem1)

                gather_row(h1, ob1)
                fire_out(h1, ob1, osem1)
                return carry

            lax.fori_loop(0, n_pairs, body, 0)
            wait_out(hist - 2, ob0, osem0)
            wait_out(hist - 1, ob1, osem1)

    return drow_kernel


def kernel(indices, table):
    batch, hist = indices.shape
    vocab, d = table.shape
    idx_t = jnp.transpose(indices).astype(jnp.int32)
    tab_t = jnp.transpose(table)
    out_t = _make_drow_gather(hist, batch, vocab, d)(idx_t, tab_t)
    return jnp.transpose(out_t, (2, 0, 1))


# speedup vs baseline: 1.4975x; 1.0028x over previous
import functools

import jax
import jax.numpy as jnp
from jax import lax
from jax.experimental import pallas as pl
from jax.experimental.pallas import tpu as pltpu
from jax.experimental.pallas import tpu_sc as plsc

_NC = 2
_NS = 16
_NW = _NC * _NS


@functools.lru_cache(maxsize=None)
def _make_drow_gather(hist: int, batch: int, vocab: int, d: int):
    d_per_w = d // _NW
    n_pairs = hist // 2
    nv = batch // 16
    assert d % _NW == 0 and hist % 2 == 0 and batch % 16 == 0

    mesh = plsc.VectorSubcoreMesh(core_axis_name="c", subcore_axis_name="s")

    @functools.partial(
        pl.kernel,
        mesh=mesh,
        compiler_params=pltpu.CompilerParams(
            use_tc_tiling_on_sc=False, needs_layout_passes=False
        ),
        out_type=jax.ShapeDtypeStruct((hist, d, batch), jnp.float32),
        scratch_types=[
            pltpu.VMEM((vocab,), jnp.float32),
            pltpu.VMEM((batch,), jnp.int32),
            pltpu.VMEM((batch,), jnp.float32),
            pltpu.VMEM((batch,), jnp.float32),
            pltpu.SemaphoreType.DMA,
            pltpu.SemaphoreType.DMA,
        ],
    )
    def drow_kernel(idx_hbm, tab_hbm, out_hbm, row, ibuf, ob0, ob1,
                    osem0, osem1):
        wid = lax.axis_index("s") * _NC + lax.axis_index("c")

        for dpass in range(d_per_w):
            dd = wid * d_per_w + dpass
            pltpu.sync_copy(tab_hbm.at[dd], row)

            def fire_out(h, ob, osem):
                pltpu.async_copy(ob, out_hbm.at[h, dd], osem)

            def wait_out(h, ob, osem):
                pltpu.make_async_copy(ob, out_hbm.at[h, dd], osem).wait()

            def gather_row(h, ob):
                pltpu.sync_copy(idx_hbm.at[h], ibuf)

                @plsc.parallel_loop(0, nv, unroll=16)
                def _gather(v):
                    iv = ibuf[pl.ds(v * 16, 16)]
                    ob[pl.ds(v * 16, 16)] = plsc.load_gather(row, [iv])

            def body(t, carry):
                h0 = 2 * t
                h1 = h0 + 1

                @pl.when(t >= 1)
                def _wait_prev_out0():
                    wait_out(h0 - 2, ob0, osem0)

                gather_row(h0, ob0)
                fire_out(h0, ob0, osem0)

                @pl.when(t >= 1)
                def _wait_prev_out1():
                    wait_out(h1 - 2, ob1, osem1)

                gather_row(h1, ob1)
                fire_out(h1, ob1, osem1)
                return carry

            lax.fori_loop(0, n_pairs, body, 0)
            wait_out(hist - 2, ob0, osem0)
            wait_out(hist - 1, ob1, osem1)

    return drow_kernel


def kernel(indices, table):
    batch, hist = indices.shape
    vocab, d = table.shape
    idx_t = jnp.transpose(indices).astype(jnp.int32)
    tab_t = jnp.transpose(table)
    out_t = _make_drow_gather(hist, batch, vocab, d)(idx_t, tab_t)
    return jnp.transpose(out_t, (2, 0, 1))


# speedup vs baseline: 1.6720x; 1.1165x over previous
import functools

import jax
import jax.numpy as jnp
from jax import lax
from jax.experimental import pallas as pl
from jax.experimental.pallas import tpu as pltpu
from jax.experimental.pallas import tpu_sc as plsc

_NC = 2
_NS = 16
_NW = _NC * _NS


@functools.lru_cache(maxsize=None)
def _make_gather(batch: int, hist: int, d: int, bchunk: int):
    b_per_w = batch // _NW
    rows_per_w = b_per_w * hist
    n_groups = b_per_w // bchunk
    chunk = bchunk * hist
    n_pairs = n_groups // 2
    assert batch % _NW == 0 and n_groups % 2 == 0

    mesh = plsc.VectorSubcoreMesh(core_axis_name="c", subcore_axis_name="s")

    @functools.partial(
        pl.kernel,
        mesh=mesh,
        compiler_params=pltpu.CompilerParams(use_tc_tiling_on_sc=False),
        out_type=jax.ShapeDtypeStruct((batch, hist, d), jnp.float32),
        scratch_types=[
            pltpu.VMEM((rows_per_w,), jnp.int32),
            pltpu.VMEM((chunk, d), jnp.float32),
            pltpu.VMEM((chunk, d), jnp.float32),
            pltpu.SemaphoreType.DMA,
            pltpu.SemaphoreType.DMA,
            pltpu.SemaphoreType.DMA,
            pltpu.SemaphoreType.DMA,
        ],
    )
    def gather_kernel(idx_hbm, table_hbm, out_hbm, idx_v, rows0, rows1,
                      gsem0, gsem1, osem0, osem1):
        wid = lax.axis_index("s") * _NC + lax.axis_index("c")
        base = wid * rows_per_w
        b_base = wid * b_per_w

        def fire_gather(g, rows_v, gsem):
            pltpu.async_copy(
                table_hbm.at[idx_v.at[pl.ds(g * chunk, chunk)]],
                rows_v,
                gsem,
            )

        def wait_gather(g, rows_v, gsem):
            pltpu.make_async_copy(
                table_hbm.at[idx_v.at[pl.ds(g * chunk, chunk)]],
                rows_v,
                gsem,
            ).wait()

        def fire_out(g, rows_v, osem):
            for j in range(bchunk):
                pltpu.async_copy(
                    rows_v.at[pl.ds(j * hist, hist)],
                    out_hbm.at[b_base + g * bchunk + j],
                    osem,
                )

        def wait_out(g, rows_v, osem):
            for j in range(bchunk):
                pltpu.make_async_copy(
                    rows_v.at[pl.ds(j * hist, hist)],
                    out_hbm.at[b_base + g * bchunk + j],
                    osem,
                ).wait()

        pltpu.sync_copy(idx_hbm.at[pl.ds(base, rows_per_w)], idx_v)
        fire_gather(0, rows0, gsem0)

        def body(t, carry):
            g0 = 2 * t
            g1 = g0 + 1

            @pl.when(t >= 1)
            def _wait_prev_out1():
                wait_out(g1 - 2, rows1, osem1)

            fire_gather(g1, rows1, gsem1)
            wait_gather(g0, rows0, gsem0)
            fire_out(g0, rows0, osem0)

            @pl.when(t < n_pairs - 1)
            def _refill_buf0():
                wait_out(g0, rows0, osem0)
                fire_gather(g0 + 2, rows0, gsem0)

            wait_gather(g1, rows1, gsem1)
            fire_out(g1, rows1, osem1)
            return carry

        lax.fori_loop(0, n_pairs, body, 0)
        wait_out(n_groups - 2, rows0, osem0)
        wait_out(n_groups - 1, rows1, osem1)

    return gather_kernel


def kernel(indices, table):
    batch, hist = indices.shape
    d = table.shape[1]
    flat = indices.reshape(-1).astype(jnp.int32)
    return _make_gather(batch, hist, d, 4)(flat, table)


# speedup vs baseline: 2.0526x; 1.2276x over previous
import functools

import jax
import jax.numpy as jnp
from jax import lax
from jax.experimental import pallas as pl
from jax.experimental.pallas import tpu as pltpu
from jax.experimental.pallas import tpu_sc as plsc

_NC = 2
_NS = 16
_NW = _NC * _NS


@functools.lru_cache(maxsize=None)
def _make_drow_gather(hist: int, batch: int, vocab: int, d: int):
    d_per_w = d // _NW
    n_pairs = hist // 2
    nv = batch // 16
    assert d % _NW == 0 and hist % 2 == 0 and batch % 16 == 0

    mesh = plsc.VectorSubcoreMesh(core_axis_name="c", subcore_axis_name="s")

    @functools.partial(
        pl.kernel,
        mesh=mesh,
        compiler_params=pltpu.CompilerParams(
            use_tc_tiling_on_sc=False, needs_layout_passes=False
        ),
        out_type=jax.ShapeDtypeStruct((hist, d, batch), jnp.float32),
        scratch_types=[
            pltpu.VMEM((vocab,), jnp.float32),
            pltpu.VMEM((batch,), jnp.int32),
            pltpu.VMEM((batch,), jnp.int32),
            pltpu.VMEM((batch,), jnp.float32),
            pltpu.VMEM((batch,), jnp.float32),
            pltpu.SemaphoreType.DMA,
            pltpu.SemaphoreType.DMA,
            pltpu.SemaphoreType.DMA,
            pltpu.SemaphoreType.DMA,
        ],
    )
    def drow_kernel(idx_hbm, tab_hbm, out_hbm, row, ib0, ib1, ob0, ob1,
                    isem0, isem1, osem0, osem1):
        wid = lax.axis_index("s") * _NC + lax.axis_index("c")

        def fire_idx(h, ib, isem):
            pltpu.async_copy(idx_hbm.at[h], ib, isem)

        def wait_idx(h, ib, isem):
            pltpu.make_async_copy(idx_hbm.at[h], ib, isem).wait()

        for dpass in range(d_per_w):
            dd = wid * d_per_w + dpass
            pltpu.sync_copy(tab_hbm.at[dd], row)

            def fire_out(h, ob, osem):
                pltpu.async_copy(ob, out_hbm.at[h, dd], osem)

            def wait_out(h, ob, osem):
                pltpu.make_async_copy(ob, out_hbm.at[h, dd], osem).wait()

            def gather_row(ib, ob):
                @plsc.parallel_loop(0, nv, unroll=8)
                def _gather(v):
                    iv = ib[pl.ds(v * 16, 16)]
                    ob[pl.ds(v * 16, 16)] = plsc.load_gather(row, [iv])

            fire_idx(0, ib0, isem0)
            fire_idx(1, ib1, isem1)

            def body(t, carry):
                h0 = 2 * t
                h1 = h0 + 1

                @pl.when(t >= 1)
                def _wait_prev_out0():
                    wait_out(h0 - 2, ob0, osem0)

                wait_idx(h0, ib0, isem0)
                gather_row(ib0, ob0)
                fire_out(h0, ob0, osem0)

                @pl.when(t < n_pairs - 1)
                def _prefetch_idx0():
                    fire_idx(h0 + 2, ib0, isem0)

                @pl.when(t >= 1)
                def _wait_prev_out1():
                    wait_out(h1 - 2, ob1, osem1)

                wait_idx(h1, ib1, isem1)
                gather_row(ib1, ob1)
                fire_out(h1, ob1, osem1)

                @pl.when(t < n_pairs - 1)
                def _prefetch_idx1():
                    fire_idx(h1 + 2, ib1, isem1)

                return carry

            lax.fori_loop(0, n_pairs, body, 0)
            wait_out(hist - 2, ob0, osem0)
            wait_out(hist - 1, ob1, osem1)

    return drow_kernel


def kernel(indices, table):
    batch, hist = indices.shape
    vocab, d = table.shape
    idx_t = jnp.transpose(indices).astype(jnp.int32)
    tab_t = jnp.transpose(table)
    out_t = _make_drow_gather(hist, batch, vocab, d)(idx_t, tab_t)
    return jnp.transpose(out_t, (2, 0, 1))


# speedup vs baseline: 2.0583x; 1.0028x over previous
import functools

import jax
import jax.numpy as jnp
from jax import lax
from jax.experimental import pallas as pl
from jax.experimental.pallas import tpu as pltpu
from jax.experimental.pallas import tpu_sc as plsc

_NC = 2
_NS = 16
_NW = _NC * _NS


@functools.lru_cache(maxsize=None)
def _make_drow_gather(hist: int, batch: int, vocab: int, d: int):
    d_per_w = d // _NW
    n_pairs = hist // 2
    nv = batch // 16
    assert d % _NW == 0 and hist % 2 == 0 and batch % 16 == 0

    mesh = plsc.VectorSubcoreMesh(core_axis_name="c", subcore_axis_name="s")

    @functools.partial(
        pl.kernel,
        mesh=mesh,
        compiler_params=pltpu.CompilerParams(
            use_tc_tiling_on_sc=False, needs_layout_passes=False
        ),
        out_type=jax.ShapeDtypeStruct((hist, d, batch), jnp.float32),
        scratch_types=[
            pltpu.VMEM((vocab,), jnp.float32),
            pltpu.VMEM((batch,), jnp.int32),
            pltpu.VMEM((batch,), jnp.int32),
            pltpu.VMEM((batch,), jnp.float32),
            pltpu.VMEM((batch,), jnp.float32),
            pltpu.SemaphoreType.DMA,
            pltpu.SemaphoreType.DMA,
            pltpu.SemaphoreType.DMA,
            pltpu.SemaphoreType.DMA,
        ],
    )
    def drow_kernel(idx_hbm, tab_hbm, out_hbm, row, ib0, ib1, ob0, ob1,
                    isem0, isem1, osem0, osem1):
        wid = lax.axis_index("s") * _NC + lax.axis_index("c")

        def fire_idx(h, ib, isem):
            pltpu.async_copy(idx_hbm.at[h], ib, isem)

        def wait_idx(h, ib, isem):
            pltpu.make_async_copy(idx_hbm.at[h], ib, isem).wait()

        for dpass in range(d_per_w):
            dd = wid * d_per_w + dpass
            pltpu.sync_copy(tab_hbm.at[dd], row)

            def fire_out(h, ob, osem):
                pltpu.async_copy(ob, out_hbm.at[h, dd], osem)

            def wait_out(h, ob, osem):
                pltpu.make_async_copy(ob, out_hbm.at[h, dd], osem).wait()

            def gather_row(ib, ob):
                @plsc.parallel_loop(0, nv, unroll=16)
                def _gather(v):
                    iv = ib[pl.ds(v * 16, 16)]
                    ob[pl.ds(v * 16, 16)] = plsc.load_gather(row, [iv])

            fire_idx(0, ib0, isem0)
            fire_idx(1, ib1, isem1)

            def body(t, carry):
                h0 = 2 * t
                h1 = h0 + 1

                @pl.when(t >= 1)
                def _wait_prev_out0():
                    wait_out(h0 - 2, ob0, osem0)

                wait_idx(h0, ib0, isem0)
                gather_row(ib0, ob0)
                fire_out(h0, ob0, osem0)

                @pl.when(t < n_pairs - 1)
                def _prefetch_idx0():
                    fire_idx(h0 + 2, ib0, isem0)

                @pl.when(t >= 1)
                def _wait_prev_out1():
                    wait_out(h1 - 2, ob1, osem1)

                wait_idx(h1, ib1, isem1)
                gather_row(ib1, ob1)
                fire_out(h1, ob1, osem1)

                @pl.when(t < n_pairs - 1)
                def _prefetch_idx1():
                    fire_idx(h1 + 2, ib1, isem1)

                return carry

            lax.fori_loop(0, n_pairs, body, 0)
            wait_out(hist - 2, ob0, osem0)
            wait_out(hist - 1, ob1, osem1)

    return drow_kernel


def kernel(indices, table):
    batch, hist = indices.shape
    vocab, d = table.shape
    idx_t = jnp.transpose(indices).astype(jnp.int32)
    tab_t = jnp.transpose(table)
    out_t = _make_drow_gather(hist, batch, vocab, d)(idx_t, tab_t)
    return jnp.transpose(out_t, (2, 0, 1))
